# Initial kernel scaffold; baseline (speedup 1.0000x reference)
import jax
import jax.numpy as jnp
from jax.experimental import pallas as pl


def kernel(x, edge_index, W_fc, b_fc, W_c1, b_c1, g_ln1, b_ln1, W_c2, b_c2, g_ln2, b_ln2, W_p1, v_p1, W_p0, v_p0, W_f, b_f):
    raise NotImplementedError("write your pallas kernel here")


# speedup vs baseline: 4.7847x; 4.7847x over previous
import jax
import jax.numpy as jnp
from jax import lax
from jax.experimental import pallas as pl
from jax.experimental.pallas import tpu as pltpu
from jax.experimental.pallas import tpu_sc as plsc

N = 10000
E = 320000
D = 128
DH = 64
NC = 2
NS = 16
CB = 128
EPT = 19968
RPT = 624
RB = 1000
NBLK = N // RB


def _make_agg(compute_deg):
  out_type = [jax.ShapeDtypeStruct((NC, N, DH), jnp.float32)]
  scratch = [
      pltpu.VMEM_SHARED((N, DH), jnp.float32),
      pltpu.VMEM_SHARED((N, DH), jnp.float32),
      pltpu.VMEM((CB,), jnp.int32),
      pltpu.VMEM((CB,), jnp.int32),
      pltpu.VMEM((CB, DH), jnp.float32),
  ]
  if compute_deg:
    out_type.append(jax.ShapeDtypeStruct((N,), jnp.float32))
    scratch += [
        pltpu.VMEM_SHARED((N,), jnp.float32),
        pltpu.VMEM((CB,), jnp.float32),
        pltpu.VMEM((640,), jnp.float32),
    ]
  mesh = plsc.VectorSubcoreMesh(
      core_axis_name="c", subcore_axis_name="s",
      num_cores=NC, num_subcores=NS)

  def body(*refs):
    if compute_deg:
      (h_hbm, src_hbm, dst_hbm, z2_hbm, agg_hbm, deg_hbm,
       shared_h, shared_agg, idx_s, idx_d, rows,
       shared_deg, ones_v, dbuf) = refs
    else:
      (h_hbm, src_hbm, dst_hbm, z2_hbm, agg_hbm,
       shared_h, shared_agg, idx_s, idx_d, rows) = refs
    c = lax.axis_index("c")
    s = lax.axis_index("s")

    def _rowcopy(mk_src, mk_dst):
      @pl.when(s < NS - 1)
      def _():
        pltpu.sync_copy(mk_src(s * RPT, RPT), mk_dst(s * RPT, RPT))
      @pl.when(s == NS - 1)
      def _():
        pltpu.sync_copy(mk_src(15 * RPT, 640), mk_dst(15 * RPT, 640))

    _rowcopy(lambda o, n: h_hbm.at[c, pl.ds(o, n)],
             lambda o, n: shared_h.at[pl.ds(o, n)])
    _rowcopy(lambda o, n: z2_hbm.at[pl.ds(o, n)],
             lambda o, n: shared_agg.at[pl.ds(o, n)])
    if compute_deg:
      for i in range(640 // 16):
        dbuf[pl.ds(i * 16, 16)] = jnp.zeros((16,), jnp.float32)
      @pl.when(jnp.logical_and(c == 0, s < NS - 1))
      def _():
        pltpu.sync_copy(dbuf, shared_deg.at[pl.ds(s * 640, 640)])
      @pl.when(jnp.logical_and(c == 0, s == NS - 1))
      def _():
        pltpu.sync_copy(dbuf.at[pl.ds(0, 400)],
                        shared_deg.at[pl.ds(9600, 400)])
      for i in range(CB // 16):
        ones_v[pl.ds(i * 16, 16)] = jnp.full((16,), 1.0, jnp.float32)
    plsc.subcore_barrier()

    base = s * EPT
    nch = jnp.where(s == NS - 1, 160, 156)

    def step(i, carry):
      off = base + i * CB
      pltpu.sync_copy(src_hbm.at[pl.ds(off, CB)], idx_s)
      pltpu.sync_copy(dst_hbm.at[pl.ds(off, CB)], idx_d)
      pltpu.sync_copy(shared_h.at[idx_s], rows)
      pltpu.sync_copy(rows, shared_agg.at[idx_d], add=True)
      if compute_deg:
        @pl.when(c == 0)
        def _():
          pltpu.sync_copy(ones_v, shared_deg.at[idx_d], add=True)
      return carry

    lax.fori_loop(0, nch, step, 0)
    plsc.subcore_barrier()
    _rowcopy(lambda o, n: shared_agg.at[pl.ds(o, n)],
             lambda o, n: agg_hbm.at[c, pl.ds(o, n)])
    if compute_deg:
      @pl.when(jnp.logical_and(c == 0, s < NS - 1))
      def _():
        pltpu.sync_copy(shared_deg.at[pl.ds(s * 640, 640)], dbuf)
        pltpu.sync_copy(dbuf, deg_hbm.at[pl.ds(s * 640, 640)])
      @pl.when(jnp.logical_and(c == 0, s == NS - 1))
      def _():
        pltpu.sync_copy(shared_deg.at[pl.ds(9600, 400)],
                        dbuf.at[pl.ds(0, 400)])
        pltpu.sync_copy(dbuf.at[pl.ds(0, 400)],
                        deg_hbm.at[pl.ds(9600, 400)])

  return pl.kernel(body, out_type=out_type, mesh=mesh, scratch_types=scratch)


_agg_cache = {}


def _agg_with_deg(*args):
  if True not in _agg_cache:
    _agg_cache[True] = _make_agg(True)
  return _agg_cache[True](*args)


def _agg_only(*args):
  if False not in _agg_cache:
    _agg_cache[False] = _make_agg(False)
  return _agg_cache[False](*args)


def _a_body(x_ref, w_ref, b_ref, h_ref, hs_ref):
  h = jnp.dot(x_ref[...], w_ref[...],
              preferred_element_type=jnp.float32) + b_ref[...]
  h_ref[...] = h
  hs_ref[0] = h[:, :DH]
  hs_ref[1] = h[:, DH:]


def _tc_a(x, W, b):
  return pl.pallas_call(
      _a_body,
      grid=(NBLK,),
      in_specs=[
          pl.BlockSpec((RB, D), lambda i: (i, 0)),
          pl.BlockSpec((D, D), lambda i: (0, 0)),
          pl.BlockSpec((1, D), lambda i: (0, 0)),
      ],
      out_specs=[
          pl.BlockSpec((RB, D), lambda i: (i, 0)),
          pl.BlockSpec((2, RB, DH), lambda i: (0, i, 0)),
      ],
      out_shape=[
          jax.ShapeDtypeStruct((N, D), jnp.float32),
          jax.ShapeDtypeStruct((2, N, DH), jnp.float32),
      ],
  )(x, W, b)


def _conv_post(agg_blk0, agg_blk1, deg_blk, w, b, g, bb):
  a = jnp.concatenate([agg_blk0, agg_blk1], axis=-1)
  d = jnp.maximum(deg_blk, 1.0)
  y = jnp.dot(a / d, w, preferred_element_type=jnp.float32) + b
  y = jnp.where(y >= 0, y, 0.1 * y)
  m = jnp.mean(y, axis=-1, keepdims=True)
  v = jnp.mean((y - m) ** 2, axis=-1, keepdims=True)
  return (y - m) * lax.rsqrt(v + 1e-5) * g + bb


def _b_body(agg_ref, deg_ref, w_ref, b_ref, g_ref, bb_ref, h_ref, hs_ref):
  y = _conv_post(agg_ref[0], agg_ref[1], deg_ref[...],
                 w_ref[...], b_ref[...], g_ref[...], bb_ref[...])
  h_ref[...] = y
  hs_ref[0] = y[:, :DH]
  hs_ref[1] = y[:, DH:]


def _tc_b(agg, deg, W, b, g, bb):
  return pl.pallas_call(
      _b_body,
      grid=(NBLK,),
      in_specs=[
          pl.BlockSpec((2, RB, DH), lambda i: (0, i, 0)),
          pl.BlockSpec((RB, 1), lambda i: (i, 0)),
          pl.BlockSpec((D, D), lambda i: (0, 0)),
          pl.BlockSpec((1, D), lambda i: (0, 0)),
          pl.BlockSpec((1, D), lambda i: (0, 0)),
          pl.BlockSpec((1, D), lambda i: (0, 0)),
      ],
      out_specs=[
          pl.BlockSpec((RB, D), lambda i: (i, 0)),
          pl.BlockSpec((2, RB, DH), lambda i: (0, i, 0)),
      ],
      out_shape=[
          jax.ShapeDtypeStruct((N, D), jnp.float32),
          jax.ShapeDtypeStruct((2, N, DH), jnp.float32),
      ],
  )(agg, deg, W, b, g, bb)


def _c_body(agg_ref, deg_ref, h1_ref, h0_ref, wc_ref, bc_ref, g_ref, bb_ref,
            wp1_ref, vp1_ref, wp0_ref, vp0_ref, wf_ref, bf_ref,
            out_ref, hf, h0f, s01):
  i = pl.program_id(0)

  @pl.when(i < NBLK)
  def _():
    y = _conv_post(agg_ref[0], agg_ref[1], deg_ref[...],
                   wc_ref[...], bc_ref[...], g_ref[...], bb_ref[...])
    h = h1_ref[...] + y
    h0 = h0_ref[...]
    hf[pl.ds(i * RB, RB)] = h
    h0f[pl.ds(i * RB, RB)] = h0
    s1 = jnp.dot(jnp.tanh(jnp.dot(h, wp1_ref[...],
                                  preferred_element_type=jnp.float32)),
                 vp1_ref[...], preferred_element_type=jnp.float32)
    s0 = jnp.dot(jnp.tanh(jnp.dot(h0, wp0_ref[...],
                                  preferred_element_type=jnp.float32)),
                 vp0_ref[...], preferred_element_type=jnp.float32)
    s01[pl.ds(i * RB, RB)] = jnp.concatenate([s1, s0], axis=-1)

  @pl.when(i == NBLK)
  def _():
    sc = s01[...]
    mx = jnp.max(sc, axis=0, keepdims=True)
    e = jnp.exp(sc - mx)
    z = jnp.sum(e, axis=0, keepdims=True)
    dn = (((0,), (0,)), ((), ()))
    p1 = lax.dot_general(e[:, 0:4], hf[...], dn,
                         preferred_element_type=jnp.float32)
    p0 = lax.dot_general(e[:, 4:8], h0f[...], dn,
                         preferred_element_type=jnp.float32)
    z1 = jnp.reshape(z[:, 0:4], (4, 1))
    z0 = jnp.reshape(z[:, 4:8], (4, 1))
    pooled = (jnp.mean(p1 / z1, axis=0, keepdims=True)
              + jnp.mean(p0 / z0, axis=0, keepdims=True))
    out_ref[...] = jnp.dot(pooled, wf_ref[...],
                           preferred_element_type=jnp.float32) + bf_ref[...]


def _tc_c(agg, deg, h1, h0, Wc, bc, g, bb, Wp1, vp1, Wp0, vp0, Wf, bf):
  blk = lambda i: (jnp.minimum(i, NBLK - 1), 0)
  blk3 = lambda i: (0, jnp.minimum(i, NBLK - 1), 0)
  full = lambda i: (0, 0)
  return pl.pallas_call(
      _c_body,
      grid=(NBLK + 1,),
      in_specs=[
          pl.BlockSpec((2, RB, DH), blk3),
          pl.BlockSpec((RB, 1), blk),
          pl.BlockSpec((RB, D), blk),
          pl.BlockSpec((RB, D), blk),
          pl.BlockSpec((D, D), full),
          pl.BlockSpec((1, D), full),
          pl.BlockSpec((1, D), full),
          pl.BlockSpec((1, D), full),
          pl.BlockSpec((D, D), full),
          pl.BlockSpec((D, 4), full),
          pl.BlockSpec((D, D), full),
          pl.BlockSpec((D, 4), full),
          pl.BlockSpec((D, 1), full),
          pl.BlockSpec((1, 1), full),
      ],
      out_specs=pl.BlockSpec((1, 1), full),
      out_shape=jax.ShapeDtypeStruct((1, 1), jnp.float32),
      scratch_shapes=[
          pltpu.VMEM((N, D), jnp.float32),
          pltpu.VMEM((N, D), jnp.float32),
          pltpu.VMEM((N, 8), jnp.float32),
      ],
  )(agg, deg, h1, h0, Wc, bc, g, bb, Wp1, vp1, Wp0, vp0, Wf, bf)


def kernel(x, edge_index, W_fc, b_fc, W_c1, b_c1, g_ln1, b_ln1,
           W_c2, b_c2, g_ln2, b_ln2, W_p1, v_p1, W_p0, v_p0, W_f, b_f):
  src = edge_index[0].astype(jnp.int32)
  dst = edge_index[1].astype(jnp.int32)
  z2 = jnp.zeros((N, DH), jnp.float32)

  h0, h0s = _tc_a(x, W_fc, b_fc.reshape(1, D))
  agg1, deg = _agg_with_deg(h0s, src, dst, z2)
  degc = deg.reshape(N, 1)
  h1, h1s = _tc_b(agg1, degc, W_c1, b_c1.reshape(1, D),
                  g_ln1.reshape(1, D), b_ln1.reshape(1, D))
  (agg2,) = _agg_only(h1s, src, dst, z2)
  out = _tc_c(agg2, degc, h1, h0, W_c2, b_c2.reshape(1, D),
              g_ln2.reshape(1, D), b_ln2.reshape(1, D),
              W_p1, v_p1, W_p0, v_p0, W_f, b_f.reshape(1, 1))
  return out
